# Initial kernel scaffold; baseline (speedup 1.0000x reference)
import jax
import jax.numpy as jnp
from jax.experimental import pallas as pl


def kernel(x, W1, b1, W2, b2, batch_size):
    raise NotImplementedError("write your pallas kernel here")


# speedup vs baseline: 1.3182x; 1.3182x over previous
import functools

import jax
import jax.numpy as jnp
import numpy as np
from jax.experimental import pallas as pl
from jax.experimental.pallas import tpu as pltpu

_V_TILE = 1024
_TINY = float(np.finfo(np.float32).tiny)
_SPAN = float(np.float32(1.0) - np.float32(_TINY))

_K0 = 0
_K1 = 42
_K2 = _K0 ^ _K1 ^ 0x1BD11BDA
_ROT_A = (13, 15, 26, 6)
_ROT_B = (17, 29, 16, 24)


def _rotl(x, r):
    return (x << jnp.uint32(r)) | (x >> jnp.uint32(32 - r))


def _threefry_bits(flat_u32):
    ks = (jnp.uint32(_K0), jnp.uint32(_K1), jnp.uint32(_K2))
    x0 = jnp.zeros_like(flat_u32) + ks[0]
    x1 = flat_u32 + ks[1]
    rots = (_ROT_A, _ROT_B)
    for i in range(5):
        for r in rots[i % 2]:
            x0 = x0 + x1
            x1 = _rotl(x1, r)
            x1 = x1 ^ x0
        x0 = x0 + ks[(i + 1) % 3]
        x1 = x1 + ks[(i + 2) % 3] + jnp.uint32(i + 1)
    return x0 ^ x1


def _fused_kernel(n_total, n_tiles,
                  x_ref, w1_ref, b1_ref, w2_ref, b2_ref,
                  sample_ref, logp_ref,
                  h_scr, m_scr, s_scr, bestv_scr, bidx_scr, blog_scr):
    t = pl.program_id(0)
    b = x_ref.shape[0]
    v = w2_ref.shape[0]
    neg_inf = jnp.float32(-jnp.inf)

    @pl.when(t == 0)
    def _init():
        h = jax.lax.dot_general(
            x_ref[...], w1_ref[...], (((1,), (1,)), ((), ())),
            preferred_element_type=jnp.float32)
        h_scr[...] = jnp.maximum(h + b1_ref[...], 0.0)
        m_scr[...] = jnp.full((b, 1), neg_inf, jnp.float32)
        s_scr[...] = jnp.zeros((b, 1), jnp.float32)
        bestv_scr[...] = jnp.full((b, 1), neg_inf, jnp.float32)
        bidx_scr[...] = jnp.zeros((b, 1), jnp.int32)
        blog_scr[...] = jnp.zeros((b, 1), jnp.float32)

    logits = jax.lax.dot_general(
        h_scr[...], w2_ref[...], (((1,), (1,)), ((), ())),
        preferred_element_type=jnp.float32) + b2_ref[...]

    col = jax.lax.broadcasted_iota(jnp.int32, (b, v), 1) + t * v
    row = jax.lax.broadcasted_iota(jnp.int32, (b, v), 0)
    valid = col < n_total

    flat = (row * n_total + col).astype(jnp.uint32)
    bits = _threefry_bits(flat)

    fb = (bits >> jnp.uint32(9)) | jnp.uint32(0x3F800000)
    f = jax.lax.bitcast_convert_type(fb, jnp.float32) - jnp.float32(1.0)
    u = jnp.maximum(jnp.float32(_TINY),
                    f * jnp.float32(_SPAN) + jnp.float32(_TINY))
    g = -jnp.log(-jnp.log(u))

    a = jnp.where(valid, logits, neg_inf)
    pert = jnp.where(valid, g + logits, neg_inf)

    tmax = jnp.max(a, axis=1, keepdims=True)
    m_old = m_scr[...]
    m_new = jnp.maximum(m_old, tmax)
    tsum = jnp.sum(jnp.exp(a - m_new), axis=1, keepdims=True)
    s_scr[...] = s_scr[...] * jnp.exp(m_old - m_new) + tsum
    m_scr[...] = m_new

    pmax = jnp.max(pert, axis=1, keepdims=True)
    is_max = pert == pmax
    pidx = jnp.min(jnp.where(is_max, col, jnp.int32(2**30)),
                   axis=1, keepdims=True)
    logit_at = jnp.sum(jnp.where(col == pidx, a, 0.0), axis=1, keepdims=True)

    upd = pmax > bestv_scr[...]
    bestv_scr[...] = jnp.where(upd, pmax, bestv_scr[...])
    bidx_scr[...] = jnp.where(upd, pidx, bidx_scr[...])
    blog_scr[...] = jnp.where(upd, logit_at, blog_scr[...])

    @pl.when(t == n_tiles - 1)
    def _finish():
        sample_ref[...] = bidx_scr[...]
        logp_ref[...] = (blog_scr[...] - m_scr[...]) - jnp.log(s_scr[...])


def kernel(x, W1, b1, W2, b2, batch_size=1):
    bsz, e = x.shape
    h_dim = W1.shape[0]
    n = W2.shape[0]
    n_tiles = (n + _V_TILE - 1) // _V_TILE

    b1r = b1.reshape(1, h_dim)
    b2r = b2.reshape(1, n)

    grid = (n_tiles,)
    sample2d, logp2d = pl.pallas_call(
        functools.partial(_fused_kernel, n, n_tiles),
        grid=grid,
        in_specs=[
            pl.BlockSpec((bsz, e), lambda t: (0, 0)),
            pl.BlockSpec((h_dim, e), lambda t: (0, 0)),
            pl.BlockSpec((1, h_dim), lambda t: (0, 0)),
            pl.BlockSpec((_V_TILE, h_dim), lambda t: (t, 0)),
            pl.BlockSpec((1, _V_TILE), lambda t: (0, t)),
        ],
        out_specs=[
            pl.BlockSpec((bsz, 1), lambda t: (0, 0)),
            pl.BlockSpec((bsz, 1), lambda t: (0, 0)),
        ],
        out_shape=[
            jax.ShapeDtypeStruct((bsz, 1), jnp.int32),
            jax.ShapeDtypeStruct((bsz, 1), jnp.float32),
        ],
        scratch_shapes=[
            pltpu.VMEM((bsz, h_dim), jnp.float32),
            pltpu.VMEM((bsz, 1), jnp.float32),
            pltpu.VMEM((bsz, 1), jnp.float32),
            pltpu.VMEM((bsz, 1), jnp.float32),
            pltpu.VMEM((bsz, 1), jnp.int32),
            pltpu.VMEM((bsz, 1), jnp.float32),
        ],
        compiler_params=pltpu.CompilerParams(
            dimension_semantics=("arbitrary",),
        ),
    )(x, W1, b1r, W2, b2r)

    return (sample2d.reshape(bsz), logp2d.reshape(bsz))


# speedup vs baseline: 1.3214x; 1.0025x over previous
import functools

import jax
import jax.numpy as jnp
import numpy as np
from jax.experimental import pallas as pl
from jax.experimental.pallas import tpu as pltpu

_V_TILE = 2048
_B_CHUNKS = 4
_TINY = float(np.finfo(np.float32).tiny)
_SPAN = float(np.float32(1.0) - np.float32(_TINY))

_K0 = 0
_K1 = 42
_K2 = _K0 ^ _K1 ^ 0x1BD11BDA
_ROT_A = (13, 15, 26, 6)
_ROT_B = (17, 29, 16, 24)


def _rotl(x, r):
    return (x << jnp.uint32(r)) | (x >> jnp.uint32(32 - r))


def _threefry_bits(flat_u32):
    ks = (jnp.uint32(_K0), jnp.uint32(_K1), jnp.uint32(_K2))
    x0 = jnp.zeros_like(flat_u32) + ks[0]
    x1 = flat_u32 + ks[1]
    rots = (_ROT_A, _ROT_B)
    for i in range(5):
        for r in rots[i % 2]:
            x0 = x0 + x1
            x1 = _rotl(x1, r)
            x1 = x1 ^ x0
        x0 = x0 + ks[(i + 1) % 3]
        x1 = x1 + ks[(i + 2) % 3] + jnp.uint32(i + 1)
    return x0 ^ x1


def _fused_kernel(n_total, n_tiles,
                  x_ref, w1_ref, b1_ref, w2_ref, b2_ref,
                  sample_ref, logp_ref,
                  h_scr, m_scr, s_scr, bestv_scr, bidx_scr, blog_scr):
    c = pl.program_id(0)
    t = pl.program_id(1)
    b = x_ref.shape[0]
    v = w2_ref.shape[0]
    neg_inf = jnp.float32(-jnp.inf)

    @pl.when(t == 0)
    def _init():
        h = jax.lax.dot_general(
            x_ref[...], w1_ref[...], (((1,), (1,)), ((), ())),
            preferred_element_type=jnp.float32)
        h_scr[...] = jnp.maximum(h + b1_ref[...], 0.0)
        m_scr[...] = jnp.full((b, 1), neg_inf, jnp.float32)
        s_scr[...] = jnp.zeros((b, 1), jnp.float32)
        bestv_scr[...] = jnp.full((b, 1), neg_inf, jnp.float32)
        bidx_scr[...] = jnp.zeros((b, 1), jnp.int32)
        blog_scr[...] = jnp.zeros((b, 1), jnp.float32)

    logits = jax.lax.dot_general(
        h_scr[...], w2_ref[...], (((1,), (1,)), ((), ())),
        preferred_element_type=jnp.float32) + b2_ref[...]

    col = jax.lax.broadcasted_iota(jnp.int32, (b, v), 1) + t * v
    row = jax.lax.broadcasted_iota(jnp.int32, (b, v), 0) + c * b
    valid = col < n_total

    flat = (row * n_total + col).astype(jnp.uint32)
    bits = _threefry_bits(flat)

    fb = (bits >> jnp.uint32(9)) | jnp.uint32(0x3F800000)
    f = jax.lax.bitcast_convert_type(fb, jnp.float32) - jnp.float32(1.0)
    u = jnp.maximum(jnp.float32(_TINY),
                    f * jnp.float32(_SPAN) + jnp.float32(_TINY))
    g = -jnp.log(-jnp.log(u))

    a = jnp.where(valid, logits, neg_inf)
    pert = jnp.where(valid, g + logits, neg_inf)

    tmax = jnp.max(a, axis=1, keepdims=True)
    m_old = m_scr[...]
    m_new = jnp.maximum(m_old, tmax)
    tsum = jnp.sum(jnp.exp(a - m_new), axis=1, keepdims=True)
    s_scr[...] = s_scr[...] * jnp.exp(m_old - m_new) + tsum
    m_scr[...] = m_new

    pmax = jnp.max(pert, axis=1, keepdims=True)
    is_max = pert == pmax
    pidx = jnp.min(jnp.where(is_max, col, jnp.int32(2**30)),
                   axis=1, keepdims=True)
    logit_at = jnp.sum(jnp.where(col == pidx, a, 0.0), axis=1, keepdims=True)

    upd = pmax > bestv_scr[...]
    bestv_scr[...] = jnp.where(upd, pmax, bestv_scr[...])
    bidx_scr[...] = jnp.where(upd, pidx, bidx_scr[...])
    blog_scr[...] = jnp.where(upd, logit_at, blog_scr[...])

    @pl.when(t == n_tiles - 1)
    def _finish():
        sample_ref[...] = bidx_scr[...]
        logp_ref[...] = (blog_scr[...] - m_scr[...]) - jnp.log(s_scr[...])


def kernel(x, W1, b1, W2, b2, batch_size=1):
    bsz, e = x.shape
    h_dim = W1.shape[0]
    n = W2.shape[0]
    n_tiles = (n + _V_TILE - 1) // _V_TILE
    bc = bsz // _B_CHUNKS

    b1r = b1.reshape(1, h_dim)
    b2r = b2.reshape(1, n)

    grid = (_B_CHUNKS, n_tiles)
    sample2d, logp2d = pl.pallas_call(
        functools.partial(_fused_kernel, n, n_tiles),
        grid=grid,
        in_specs=[
            pl.BlockSpec((bc, e), lambda c, t: (c, 0)),
            pl.BlockSpec((h_dim, e), lambda c, t: (0, 0)),
            pl.BlockSpec((1, h_dim), lambda c, t: (0, 0)),
            pl.BlockSpec((_V_TILE, h_dim), lambda c, t: (t, 0)),
            pl.BlockSpec((1, _V_TILE), lambda c, t: (0, t)),
        ],
        out_specs=[
            pl.BlockSpec((bc, 1), lambda c, t: (c, 0)),
            pl.BlockSpec((bc, 1), lambda c, t: (c, 0)),
        ],
        out_shape=[
            jax.ShapeDtypeStruct((bsz, 1), jnp.int32),
            jax.ShapeDtypeStruct((bsz, 1), jnp.float32),
        ],
        scratch_shapes=[
            pltpu.VMEM((bc, h_dim), jnp.float32),
            pltpu.VMEM((bc, 1), jnp.float32),
            pltpu.VMEM((bc, 1), jnp.float32),
            pltpu.VMEM((bc, 1), jnp.float32),
            pltpu.VMEM((bc, 1), jnp.int32),
            pltpu.VMEM((bc, 1), jnp.float32),
        ],
        compiler_params=pltpu.CompilerParams(
            dimension_semantics=("parallel", "arbitrary"),
        ),
    )(x, W1, b1r, W2, b2r)

    return (sample2d.reshape(bsz), logp2d.reshape(bsz))
